# baseline (speedup 1.0000x reference)
import jax
import jax.numpy as jnp
from jax.experimental import pallas as pl
from jax.experimental.pallas import tpu as pltpu

_NB = 4
_NO = 32
_TM = 1024
_TK = 2048


def _h_kernel(af_ref, wt_ref, b_ref, out_ref):
    out_ref[0] = (
        jnp.dot(af_ref[...], wt_ref[0], preferred_element_type=jnp.float32)
        + b_ref[0]
    ).astype(out_ref.dtype)


def _mm_kernel(bi_ref, h_ref, out_ref):
    k = pl.program_id(1)

    @pl.when(k == 0)
    def _():
        out_ref[...] = jnp.zeros_like(out_ref)

    out_ref[...] += jax.lax.dot_general(
        bi_ref[...],
        h_ref[...],
        (((1,), (0,)), ((), ())),
        precision=jax.lax.Precision.DEFAULT,
        preferred_element_type=jnp.float32,
    )


def kernel(atom_features, bond_info, W, b):
    n, f = atom_features.shape
    wt = W.reshape(_NB, _NO, f).transpose(0, 2, 1)
    b2 = b.reshape(_NB, 1, _NO)

    h3 = pl.pallas_call(
        _h_kernel,
        grid=(_NB,),
        in_specs=[
            pl.BlockSpec((n, f), lambda t: (0, 0)),
            pl.BlockSpec((1, f, _NO), lambda t: (t, 0, 0)),
            pl.BlockSpec((1, 1, _NO), lambda t: (t, 0, 0)),
        ],
        out_specs=pl.BlockSpec((1, n, _NO), lambda t: (t, 0, 0)),
        out_shape=jax.ShapeDtypeStruct((_NB, n, _NO), jnp.float32),
    )(atom_features, wt, b2)
    h2 = h3.reshape(_NB * n, _NO)

    grid = (n // _TM, (_NB * n) // _TK)
    out = pl.pallas_call(
        _mm_kernel,
        grid=grid,
        in_specs=[
            pl.BlockSpec((_TM, _TK), lambda i, k: (i, k)),
            pl.BlockSpec((_TK, _NO), lambda i, k: (k, 0)),
        ],
        out_specs=pl.BlockSpec((_TM, _NO), lambda i, k: (i, 0)),
        out_shape=jax.ShapeDtypeStruct((n, _NO), jnp.float32),
        compiler_params=pltpu.CompilerParams(
            dimension_semantics=("parallel", "arbitrary"),
        ),
    )(bond_info, h2)
    return out


# speedup vs baseline: 1.1068x; 1.1068x over previous
import jax
import jax.numpy as jnp
from jax.experimental import pallas as pl
from jax.experimental.pallas import tpu as pltpu

_NB = 4
_NO = 32
_TM = 1024
_TK = 2048


def _fused_kernel(af_ref, wt_ref, b_ref, bi_ref, out_ref):
    k = pl.program_id(1)
    nk_per_type = af_ref.shape[0] // _TK
    t = k // nk_per_type
    row = (k % nk_per_type) * _TK

    w_t = wt_ref[pl.ds(t, 1)][0]
    b_t = b_ref[pl.ds(t, 1)][0]
    af_blk = af_ref[pl.ds(row, _TK), :]
    h_blk = (
        jnp.dot(af_blk, w_t, preferred_element_type=jnp.float32) + b_t
    ).astype(jnp.bfloat16)

    @pl.when(k == 0)
    def _():
        out_ref[...] = jnp.zeros_like(out_ref)

    out_ref[...] += jnp.dot(
        bi_ref[...].astype(jnp.bfloat16),
        h_blk,
        preferred_element_type=jnp.float32,
    )


def kernel(atom_features, bond_info, W, b):
    n, f = atom_features.shape
    wt = W.reshape(_NB, _NO, f).transpose(0, 2, 1)
    b2 = b.reshape(_NB, 1, _NO)

    grid = (n // _TM, (_NB * n) // _TK)
    out = pl.pallas_call(
        _fused_kernel,
        grid=grid,
        in_specs=[
            pl.BlockSpec((n, f), lambda i, k: (0, 0)),
            pl.BlockSpec((_NB, f, _NO), lambda i, k: (0, 0, 0)),
            pl.BlockSpec((_NB, 1, _NO), lambda i, k: (0, 0, 0)),
            pl.BlockSpec((_TM, _TK), lambda i, k: (i, k)),
        ],
        out_specs=pl.BlockSpec((_TM, _NO), lambda i, k: (i, 0)),
        out_shape=jax.ShapeDtypeStruct((n, _NO), jnp.float32),
        compiler_params=pltpu.CompilerParams(
            dimension_semantics=("parallel", "arbitrary"),
        ),
    )(atom_features, wt, b2, bond_info)
    return out


# speedup vs baseline: 1.1295x; 1.0205x over previous
import jax
import jax.numpy as jnp
from jax.experimental import pallas as pl
from jax.experimental.pallas import tpu as pltpu

_NB = 4
_NO = 32
_TM = 1024
_TK = 1024
_NS = 2


def _fused_kernel(af_ref, wt_ref, b_ref, bi0_ref, bi1_ref, out_ref, h_ref):
    i = pl.program_id(0)
    k = pl.program_id(1)
    n = af_ref.shape[0]

    @pl.when((i == 0) & (k == 0))
    def _():
        af16 = af_ref[...].astype(jnp.bfloat16)
        for t in range(_NB):
            h_t = (
                jnp.dot(
                    af16,
                    wt_ref[t].astype(jnp.bfloat16),
                    preferred_element_type=jnp.float32,
                )
                + b_ref[t]
            )
            h_ref[pl.ds(t * n, n), :] = h_t.astype(jnp.bfloat16)

    @pl.when(k == 0)
    def _():
        out_ref[...] = jnp.zeros_like(out_ref)

    base = k * (_NS * _TK)
    acc = jnp.dot(
        bi0_ref[...].astype(jnp.bfloat16),
        h_ref[pl.ds(base, _TK), :],
        preferred_element_type=jnp.float32,
    )
    acc += jnp.dot(
        bi1_ref[...].astype(jnp.bfloat16),
        h_ref[pl.ds(base + _TK, _TK), :],
        preferred_element_type=jnp.float32,
    )
    out_ref[...] += acc


def kernel(atom_features, bond_info, W, b):
    n, f = atom_features.shape
    wt = W.reshape(_NB, _NO, f).transpose(0, 2, 1)
    b2 = b.reshape(_NB, 1, _NO)

    grid = (n // _TM, (_NB * n) // (_NS * _TK))
    out = pl.pallas_call(
        _fused_kernel,
        grid=grid,
        in_specs=[
            pl.BlockSpec((n, f), lambda i, k: (0, 0)),
            pl.BlockSpec((_NB, f, _NO), lambda i, k: (0, 0, 0)),
            pl.BlockSpec((_NB, 1, _NO), lambda i, k: (0, 0, 0)),
            pl.BlockSpec((_TM, _TK), lambda i, k: (i, 2 * k)),
            pl.BlockSpec((_TM, _TK), lambda i, k: (i, 2 * k + 1)),
        ],
        out_specs=pl.BlockSpec((_TM, _NO), lambda i, k: (i, 0)),
        out_shape=jax.ShapeDtypeStruct((n, _NO), jnp.float32),
        scratch_shapes=[pltpu.VMEM((_NB * n, _NO), jnp.bfloat16)],
        compiler_params=pltpu.CompilerParams(
            dimension_semantics=("parallel", "arbitrary"),
        ),
    )(atom_features, wt, b2, bond_info, bond_info)
    return out
